# baseline (speedup 1.0000x reference)
import jax
import jax.numpy as jnp
from jax import lax
from jax.experimental import pallas as pl
from jax.experimental.pallas import tpu as pltpu
from jax.experimental.pallas import tpu_sc as plsc

B = 16384
D = 784
NC, NS = 2, 16
RPW = D // NS
BH = B // NC
NG = BH // 16


def _body(idx_hbm, tabt_hbm, out_hbm, tabt_v, idx_v, buf0, buf1, sem0, sem1):
    rg = lax.axis_index("s")
    h = lax.axis_index("c")
    j0 = rg * RPW
    bbase = h * BH
    pltpu.sync_copy(tabt_hbm, tabt_v)
    pltpu.sync_copy(idx_hbm.at[pl.ds(bbase, BH)], idx_v)

    dnums = lax.GatherDimensionNumbers(
        offset_dims=(), collapsed_slice_dims=(0,), start_index_map=(0,)
    )

    def compute_row(j, buf):
        rowv = tabt_v[pl.ds(j * 10, 16)]

        def per_group(g, _):
            xg = idx_v[pl.ds(g * 16, 16)]
            v = lax.gather(
                rowv, xg[:, None], dnums, (1,),
                mode=lax.GatherScatterMode.PROMISE_IN_BOUNDS,
            )
            buf[pl.ds(g * 16, 16)] = v
            return 0

        lax.fori_loop(0, NG, per_group, 0, unroll=8)

    def out_slice(j):
        return out_hbm.at[pl.ds(j * B + bbase, BH)]

    def row_pair(hh, _):
        ja = j0 + 2 * hh
        jb = ja + 1

        @pl.when(hh > 0)
        def _():
            pltpu.make_async_copy(buf0, out_slice(ja - 2), sem0).wait()

        compute_row(ja, buf0)
        pltpu.async_copy(buf0, out_slice(ja), sem0)

        @pl.when(hh > 0)
        def _():
            pltpu.make_async_copy(buf1, out_slice(jb - 2), sem1).wait()

        @pl.when(jb < j0 + RPW)
        def _():
            compute_row(jb, buf1)
            pltpu.async_copy(buf1, out_slice(jb), sem1)

        return 0

    lax.fori_loop(0, (RPW + 1) // 2, row_pair, 0)
    pltpu.make_async_copy(buf0, out_slice(j0 + RPW - 1), sem0).wait()


@jax.jit
def _gather_t(x, logits):
    mesh = plsc.VectorSubcoreMesh(core_axis_name="c", subcore_axis_name="s")
    idx = x.astype(jnp.int32)
    tabt = jnp.concatenate([logits.T.reshape(D * 10), jnp.zeros((16,), jnp.float32)])
    run = pl.kernel(
        _body,
        mesh=mesh,
        out_type=jax.ShapeDtypeStruct((D * B,), jnp.float32),
        scratch_types=[
            pltpu.VMEM((D * 10 + 16,), jnp.float32),
            pltpu.VMEM((BH,), jnp.int32),
            pltpu.VMEM((BH,), jnp.float32),
            pltpu.VMEM((BH,), jnp.float32),
            pltpu.SemaphoreType.DMA,
            pltpu.SemaphoreType.DMA,
        ],
    )
    out_t = run(idx, tabt)
    return jnp.transpose(out_t.reshape(1, 28, 28, B), (3, 0, 1, 2))


def kernel(x, logits):
    return _gather_t(x, logits)


# speedup vs baseline: 2.0908x; 2.0908x over previous
import jax
import jax.numpy as jnp
from jax import lax
from jax.experimental import pallas as pl
from jax.experimental.pallas import tpu as pltpu
from jax.experimental.pallas import tpu_sc as plsc

B = 16384
D = 784
NC, NS = 2, 16
RPW = D // NS
BH = B // NC
NG = BH // 16
RQ = 4
NQ = (RPW + RQ - 1) // RQ
PAD = RQ * NQ - RPW + 2


def _body(idx_hbm, tabt_hbm, out_hbm, tabt_v, idx_v,
          buf0, buf1, buf2, buf3, sem0, sem1, sem2, sem3):
    rg = lax.axis_index("s")
    h = lax.axis_index("c")
    j0 = rg * RPW
    bbase = h * BH
    pltpu.sync_copy(tabt_hbm, tabt_v)
    pltpu.sync_copy(idx_hbm.at[pl.ds(bbase, BH)], idx_v)

    bufs = (buf0, buf1, buf2, buf3)
    sems = (sem0, sem1, sem2, sem3)
    dnums = lax.GatherDimensionNumbers(
        offset_dims=(), collapsed_slice_dims=(0,), start_index_map=(0,)
    )

    def out_slice(j):
        return out_hbm.at[pl.ds(j * B + bbase, BH)]

    def quad(q, _):
        ja = j0 + RQ * q

        for rr in range(RQ):
            @pl.when(q > 0)
            def _():
                pltpu.make_async_copy(
                    bufs[rr], out_slice(ja - RQ + rr), sems[rr]
                ).wait()

        def per_g8(g8, _):
            xs = [idx_v[pl.ds((g8 * 8 + k) * 16, 16)] for k in range(8)]
            for rr in range(RQ):
                rowv = tabt_v[pl.ds((ja + rr) * 10, 16)]
                for k in range(8):
                    v = lax.gather(
                        rowv, xs[k][:, None], dnums, (1,),
                        mode=lax.GatherScatterMode.PROMISE_IN_BOUNDS,
                    )
                    bufs[rr][pl.ds((g8 * 8 + k) * 16, 16)] = v
            return 0

        lax.fori_loop(0, NG // 8, per_g8, 0)

        for rr in range(RQ):
            @pl.when(ja + rr < j0 + RPW)
            def _():
                pltpu.async_copy(bufs[rr], out_slice(ja + rr), sems[rr])

        return 0

    lax.fori_loop(0, NQ, quad, 0)
    pltpu.make_async_copy(buf0, out_slice(j0 + RPW - 1), sem0).wait()


@jax.jit
def _gather_t(x, logits):
    mesh = plsc.VectorSubcoreMesh(core_axis_name="c", subcore_axis_name="s")
    idx = x.astype(jnp.int32)
    tabt = jnp.concatenate(
        [logits.T.reshape(D * 10), jnp.zeros((PAD * 10 + 16,), jnp.float32)]
    )
    run = pl.kernel(
        _body,
        mesh=mesh,
        out_type=jax.ShapeDtypeStruct((D * B,), jnp.float32),
        scratch_types=[
            pltpu.VMEM(((D + PAD) * 10 + 16,), jnp.float32),
            pltpu.VMEM((BH,), jnp.int32),
            pltpu.VMEM((BH,), jnp.float32),
            pltpu.VMEM((BH,), jnp.float32),
            pltpu.VMEM((BH,), jnp.float32),
            pltpu.VMEM((BH,), jnp.float32),
            pltpu.SemaphoreType.DMA,
            pltpu.SemaphoreType.DMA,
            pltpu.SemaphoreType.DMA,
            pltpu.SemaphoreType.DMA,
        ],
    )
    out_t = run(idx, tabt)
    return jnp.transpose(out_t.reshape(1, 28, 28, B), (3, 0, 1, 2))


def kernel(x, logits):
    return _gather_t(x, logits)


# speedup vs baseline: 2.1276x; 1.0176x over previous
import jax
import jax.numpy as jnp
from jax import lax
from jax.experimental import pallas as pl
from jax.experimental.pallas import tpu as pltpu
from jax.experimental.pallas import tpu_sc as plsc

B = 16384
D = 784
NC, NS = 2, 16
RPW = D // NS
BH = B // NC
NG = BH // 16
RQ = 7
NQ = RPW // RQ


def _body(idx_hbm, tabt_hbm, out_hbm, tabt_v, idx_v, *bs):
    bufs, sems = bs[:RQ], bs[RQ:]
    rg = lax.axis_index("s")
    h = lax.axis_index("c")
    j0 = rg * RPW
    bbase = h * BH
    pltpu.sync_copy(tabt_hbm, tabt_v)
    pltpu.sync_copy(idx_hbm.at[pl.ds(bbase, BH)], idx_v)

    dnums = lax.GatherDimensionNumbers(
        offset_dims=(), collapsed_slice_dims=(0,), start_index_map=(0,)
    )

    def out_slice(j):
        return out_hbm.at[pl.ds(j * B + bbase, BH)]

    def quad(q, _):
        ja = j0 + RQ * q

        for rr in range(RQ):
            @pl.when(q > 0)
            def _():
                pltpu.make_async_copy(
                    bufs[rr], out_slice(ja - RQ + rr), sems[rr]
                ).wait()

        def per_g8(g8, _):
            xs = [idx_v[pl.ds((g8 * 8 + k) * 16, 16)] for k in range(8)]
            for rr in range(RQ):
                rowv = tabt_v[pl.ds((ja + rr) * 10, 16)]
                for k in range(8):
                    v = lax.gather(
                        rowv, xs[k][:, None], dnums, (1,),
                        mode=lax.GatherScatterMode.PROMISE_IN_BOUNDS,
                    )
                    bufs[rr][pl.ds((g8 * 8 + k) * 16, 16)] = v
            return 0

        lax.fori_loop(0, NG // 8, per_g8, 0)

        for rr in range(RQ):
            pltpu.async_copy(bufs[rr], out_slice(ja + rr), sems[rr])
        return 0

    lax.fori_loop(0, NQ, quad, 0)
    for rr in range(RQ):
        pltpu.make_async_copy(
            bufs[rr], out_slice(j0 + RQ * (NQ - 1) + rr), sems[rr]
        ).wait()


@jax.jit
def _gather_t(x, logits):
    mesh = plsc.VectorSubcoreMesh(core_axis_name="c", subcore_axis_name="s")
    idx = x.astype(jnp.int32)
    tabt = jnp.concatenate([logits.T.reshape(D * 10), jnp.zeros((16,), jnp.float32)])
    run = pl.kernel(
        _body,
        mesh=mesh,
        out_type=jax.ShapeDtypeStruct((D * B,), jnp.float32),
        scratch_types=(
            [pltpu.VMEM((D * 10 + 16,), jnp.float32), pltpu.VMEM((BH,), jnp.int32)]
            + [pltpu.VMEM((BH,), jnp.float32)] * RQ
            + [pltpu.SemaphoreType.DMA] * RQ
        ),
    )
    out_t = run(idx, tabt)
    return jnp.transpose(out_t.reshape(1, 28, 28, B), (3, 0, 1, 2))


def kernel(x, logits):
    return _gather_t(x, logits)


# speedup vs baseline: 2.1435x; 1.0075x over previous
import jax
import jax.numpy as jnp
from jax import lax
from jax.experimental import pallas as pl
from jax.experimental.pallas import tpu as pltpu
from jax.experimental.pallas import tpu_sc as plsc

B = 16384
D = 784
NC, NS = 2, 16
RPW = D // NS
BH = B // NC
NG = BH // 16
RQ = 7
NQ = RPW // RQ


def _body(idx_hbm, tabt_hbm, out_hbm, tabt_v, idx_v, *bs):
    bufs, sems = bs[:RQ], bs[RQ:]
    rg = lax.axis_index("s")
    h = lax.axis_index("c")
    j0 = rg * RPW
    bbase = h * BH
    pltpu.sync_copy(tabt_hbm, tabt_v)
    pltpu.sync_copy(idx_hbm.at[pl.ds(bbase, BH)], idx_v)

    dnums = lax.GatherDimensionNumbers(
        offset_dims=(), collapsed_slice_dims=(0,), start_index_map=(0,)
    )

    def out_slice(j):
        return out_hbm.at[pl.ds(j * B + bbase, BH)]

    def quad(q, _):
        ja = j0 + RQ * q

        for rr in range(RQ):
            @pl.when(q > 0)
            def _():
                pltpu.make_async_copy(
                    bufs[rr], out_slice(ja - RQ + rr), sems[rr]
                ).wait()

        def per_g8(g8, _):
            xs = [idx_v[pl.ds((g8 * 8 + k) * 16, 16)] for k in range(8)]
            for rr in range(RQ):
                rowv = tabt_v[pl.ds((ja + rr) * 10, 16)]
                for k in range(8):
                    v = lax.gather(
                        rowv, xs[k][:, None], dnums, (1,),
                        mode=lax.GatherScatterMode.PROMISE_IN_BOUNDS,
                    )
                    bufs[rr][pl.ds((g8 * 8 + k) * 16, 16)] = v
            return 0

        lax.fori_loop(0, NG // 8, per_g8, 0, unroll=2)

        for rr in range(RQ):
            pltpu.async_copy(bufs[rr], out_slice(ja + rr), sems[rr])
        return 0

    lax.fori_loop(0, NQ, quad, 0)
    for rr in range(RQ):
        pltpu.make_async_copy(
            bufs[rr], out_slice(j0 + RQ * (NQ - 1) + rr), sems[rr]
        ).wait()


@jax.jit
def _gather_t(x, logits):
    mesh = plsc.VectorSubcoreMesh(core_axis_name="c", subcore_axis_name="s")
    idx = x.astype(jnp.int32)
    tabt = jnp.concatenate([logits.T.reshape(D * 10), jnp.zeros((16,), jnp.float32)])
    run = pl.kernel(
        _body,
        mesh=mesh,
        out_type=jax.ShapeDtypeStruct((D * B,), jnp.float32),
        scratch_types=(
            [pltpu.VMEM((D * 10 + 16,), jnp.float32), pltpu.VMEM((BH,), jnp.int32)]
            + [pltpu.VMEM((BH,), jnp.float32)] * RQ
            + [pltpu.SemaphoreType.DMA] * RQ
        ),
    )
    out_t = run(idx, tabt)
    return jnp.transpose(out_t.reshape(1, 28, 28, B), (3, 0, 1, 2))


def kernel(x, logits):
    return _gather_t(x, logits)


# speedup vs baseline: 2.6509x; 1.2367x over previous
import jax
import jax.numpy as jnp
from jax import lax
from jax.experimental import pallas as pl
from jax.experimental.pallas import tpu as pltpu
from jax.experimental.pallas import tpu_sc as plsc

B = 16384
D = 784
NC, NS = 2, 16
RPW = D // NS
BH = B // NC
NG = BH // 16
RQ = 7
NQ = RPW // RQ


def _body(idx_hbm, tabt_hbm, out_hbm, tabt_v, idx_v, *bs):
    bufs, sems = bs[:RQ], bs[RQ:]
    rg = lax.axis_index("s")
    h = lax.axis_index("c")
    j0 = rg * RPW
    bbase = h * BH
    pltpu.sync_copy(tabt_hbm, tabt_v)
    pltpu.sync_copy(idx_hbm.at[pl.ds(bbase, BH)], idx_v)

    dnums = lax.GatherDimensionNumbers(
        offset_dims=(), collapsed_slice_dims=(0,), start_index_map=(0,)
    )

    def out_slice(j):
        return out_hbm.at[pl.ds(j * B + bbase, BH)]

    def quad(q, _):
        ja = j0 + RQ * q

        for rr in range(RQ):
            @pl.when(q > 0)
            def _():
                pltpu.make_async_copy(
                    bufs[rr], out_slice(ja - RQ + rr), sems[rr]
                ).wait()

        @plsc.parallel_loop(0, NG // 8, unroll=2)
        def per_g8(g8):
            xs = [idx_v[pl.ds((g8 * 8 + k) * 16, 16)] for k in range(8)]
            for rr in range(RQ):
                rowv = tabt_v[pl.ds((ja + rr) * 10, 16)]
                for k in range(8):
                    v = lax.gather(
                        rowv, xs[k][:, None], dnums, (1,),
                        mode=lax.GatherScatterMode.PROMISE_IN_BOUNDS,
                    )
                    bufs[rr][pl.ds((g8 * 8 + k) * 16, 16)] = v

        for rr in range(RQ):
            pltpu.async_copy(bufs[rr], out_slice(ja + rr), sems[rr])
        return 0

    lax.fori_loop(0, NQ, quad, 0)
    for rr in range(RQ):
        pltpu.make_async_copy(
            bufs[rr], out_slice(j0 + RQ * (NQ - 1) + rr), sems[rr]
        ).wait()


@jax.jit
def _gather_t(x, logits):
    mesh = plsc.VectorSubcoreMesh(core_axis_name="c", subcore_axis_name="s")
    idx = x.astype(jnp.int32)
    tabt = jnp.concatenate([logits.T.reshape(D * 10), jnp.zeros((16,), jnp.float32)])
    run = pl.kernel(
        _body,
        mesh=mesh,
        out_type=jax.ShapeDtypeStruct((D * B,), jnp.float32),
        scratch_types=(
            [pltpu.VMEM((D * 10 + 16,), jnp.float32), pltpu.VMEM((BH,), jnp.int32)]
            + [pltpu.VMEM((BH,), jnp.float32)] * RQ
            + [pltpu.SemaphoreType.DMA] * RQ
        ),
    )
    out_t = run(idx, tabt)
    return jnp.transpose(out_t.reshape(1, 28, 28, B), (3, 0, 1, 2))


def kernel(x, logits):
    return _gather_t(x, logits)
